# Initial kernel scaffold; baseline (speedup 1.0000x reference)
import jax
import jax.numpy as jnp
from jax.experimental import pallas as pl


def kernel(z, pos_edge_index, neg_edge_index):
    raise NotImplementedError("write your pallas kernel here")


# speedup vs baseline: 3.2020x; 3.2020x over previous
import functools

import jax
import jax.numpy as jnp
from jax import lax
from jax.experimental import pallas as pl
from jax.experimental.pallas import tpu as pltpu
from jax.experimental.pallas import tpu_sc as plsc

LOG_EPS = 1e-8
NE = 320000
DIM = 128
LANES = 16
NWORK = 32
PER_W = NE // NWORK
CH = 128
NFULL = PER_W // CH
TAIL = PER_W - NFULL * CH


def _sc_partials(z, pos_edge_index, neg_edge_index):
  mesh = plsc.VectorSubcoreMesh(core_axis_name="c", subcore_axis_name="s")

  @functools.partial(
      pl.kernel,
      out_type=jax.ShapeDtypeStruct((2 * NE, LANES), jnp.float32),
      mesh=mesh,
      scratch_types=[
          pltpu.VMEM((CH,), jnp.int32),
          pltpu.VMEM((CH,), jnp.int32),
          pltpu.VMEM((CH, DIM), jnp.float32),
          pltpu.VMEM((CH, DIM), jnp.float32),
          pltpu.VMEM((CH, LANES), jnp.float32),
          pltpu.SemaphoreType.DMA,
          pltpu.SemaphoreType.DMA,
      ],
  )
  def k(z_hbm, pos_hbm, neg_hbm, out_hbm, sidx, didx, srows, drows, outv,
        sem1, sem2):
    wid = lax.axis_index("s") * 2 + lax.axis_index("c")

    def do_chunk(eidx_hbm, ebase, obase, n):
      pltpu.sync_copy(eidx_hbm.at[0, pl.ds(ebase, n)], sidx.at[pl.ds(0, n)])
      pltpu.sync_copy(eidx_hbm.at[1, pl.ds(ebase, n)], didx.at[pl.ds(0, n)])
      c1 = pltpu.async_copy(z_hbm.at[sidx.at[pl.ds(0, n)]],
                            srows.at[pl.ds(0, n)], sem1)
      c2 = pltpu.async_copy(z_hbm.at[didx.at[pl.ds(0, n)]],
                            drows.at[pl.ds(0, n)], sem2)
      c1.wait()
      c2.wait()

      def body(e, carry):
        acc = srows[e, 0:LANES] * drows[e, 0:LANES]
        for j in range(1, DIM // LANES):
          acc = acc + (srows[e, j * LANES:(j + 1) * LANES] *
                       drows[e, j * LANES:(j + 1) * LANES])
        outv[e, :] = acc
        return carry

      lax.fori_loop(0, n, body, 0)
      pltpu.sync_copy(outv.at[pl.ds(0, n)], out_hbm.at[pl.ds(obase, n)])

    for eidx_hbm, sign_off in ((pos_hbm, 0), (neg_hbm, NE)):
      base = wid * PER_W

      def chunk_loop(t, carry, eidx_hbm=eidx_hbm, sign_off=sign_off,
                     base=base):
        ebase = pl.multiple_of(base + t * CH, CH)
        do_chunk(eidx_hbm, ebase, sign_off + ebase, CH)
        return carry

      lax.fori_loop(0, NFULL, chunk_loop, 0)
      if TAIL:
        tbase = base + NFULL * CH
        do_chunk(eidx_hbm, tbase, sign_off + tbase, TAIL)

  return k(z, pos_edge_index, neg_edge_index)


def _loss_body(p_ref, o_ref, *, npos_blocks, rows):
  i = pl.program_id(0)
  x = p_ref[...]
  r = lax.broadcasted_iota(jnp.int32, (DIM, DIM // LANES), 0)
  c = lax.broadcasted_iota(jnp.int32, (DIM, DIM // LANES), 1)
  fold = jnp.where(r // LANES == c, 1.0, 0.0).astype(jnp.float32)
  v = jnp.dot(x, fold, preferred_element_type=jnp.float32)
  s = jax.nn.sigmoid(v)
  arg = jnp.where(i < npos_blocks, s, 1.0 - s) + LOG_EPS
  t = -jnp.log(arg)

  @pl.when(i == 0)
  def _():
    o_ref[...] = jnp.zeros((1, 1), jnp.float32)

  o_ref[...] += jnp.full((1, 1), jnp.sum(t) * (1.0 / NE), jnp.float32)


def _tc_loss(partials):
  total_rows = 2 * NE * LANES // DIM
  rows = 4000
  grid = total_rows // rows
  body = functools.partial(_loss_body, npos_blocks=grid // 2, rows=rows)
  out = pl.pallas_call(
      body,
      out_shape=jax.ShapeDtypeStruct((1, 1), jnp.float32),
      grid=(grid,),
      in_specs=[pl.BlockSpec((rows, DIM), lambda i: (i, 0))],
      out_specs=pl.BlockSpec((1, 1), lambda i: (0, 0)),
  )(partials.reshape(total_rows, DIM))
  return out[0, 0]


def kernel(z, pos_edge_index, neg_edge_index):
  partials = _sc_partials(z, pos_edge_index, neg_edge_index)
  return _tc_loss(partials)


# speedup vs baseline: 4.4878x; 1.4015x over previous
import functools

import jax
import jax.numpy as jnp
from jax import lax
from jax.experimental import pallas as pl
from jax.experimental.pallas import tpu as pltpu
from jax.experimental.pallas import tpu_sc as plsc

LOG_EPS = 1e-8
NE = 320000
DIM = 128
LANES = 16
NWORK = 32
PER_W = 2 * NE // NWORK
CH = 80
NCH = PER_W // CH


def _sc_partials(z, src_all, dst_all):
  mesh = plsc.VectorSubcoreMesh(core_axis_name="c", subcore_axis_name="s")

  @functools.partial(
      pl.kernel,
      out_type=jax.ShapeDtypeStruct((2 * NE, LANES), jnp.float32),
      mesh=mesh,
      scratch_types=[
          pltpu.VMEM((PER_W,), jnp.int32),
          pltpu.VMEM((PER_W,), jnp.int32),
          pltpu.VMEM((CH, DIM), jnp.float32),
          pltpu.VMEM((CH, DIM), jnp.float32),
          pltpu.VMEM((CH, DIM), jnp.float32),
          pltpu.VMEM((CH, DIM), jnp.float32),
          pltpu.VMEM((CH, LANES), jnp.float32),
          pltpu.VMEM((CH, LANES), jnp.float32),
          pltpu.SemaphoreType.DMA,
          pltpu.SemaphoreType.DMA,
          pltpu.SemaphoreType.DMA,
          pltpu.SemaphoreType.DMA,
      ],
  )
  def k(z_hbm, src_hbm, dst_hbm, out_hbm, sidx, didx, sr0, dr0, sr1, dr1,
        ov0, ov1, gs0, gs1, os0, os1):
    wid = lax.axis_index("s") * 2 + lax.axis_index("c")
    ebase = pl.multiple_of(wid * PER_W, PER_W)
    srows = (sr0, sr1)
    drows = (dr0, dr1)
    outv = (ov0, ov1)
    gsem = (gs0, gs1)
    osem = (os0, os1)

    pltpu.sync_copy(src_hbm.at[pl.ds(ebase, PER_W)], sidx)
    pltpu.sync_copy(dst_hbm.at[pl.ds(ebase, PER_W)], didx)

    def issue_gather(i, b):
      off = pl.multiple_of(i * CH, CH)
      pltpu.async_copy(z_hbm.at[sidx.at[pl.ds(off, CH)]], srows[b], gsem[b])
      pltpu.async_copy(z_hbm.at[didx.at[pl.ds(off, CH)]], drows[b], gsem[b])

    def wait_gather(b):
      pltpu.make_async_copy(z_hbm.at[sidx.at[pl.ds(0, CH)]], srows[b],
                            gsem[b]).wait()
      pltpu.make_async_copy(z_hbm.at[didx.at[pl.ds(0, CH)]], drows[b],
                            gsem[b]).wait()

    def issue_out(i, b):
      obase = pl.multiple_of(ebase + i * CH, CH)
      pltpu.async_copy(outv[b], out_hbm.at[pl.ds(obase, CH)], osem[b])

    def wait_out(b):
      pltpu.make_async_copy(outv[b], out_hbm.at[pl.ds(0, CH)], osem[b]).wait()

    def compute(b):
      sr, dr, ov = srows[b], drows[b], outv[b]

      def ebody(e, carry):
        acc = sr[e, 0:LANES] * dr[e, 0:LANES]
        for j in range(1, DIM // LANES):
          acc = acc + (sr[e, j * LANES:(j + 1) * LANES] *
                       dr[e, j * LANES:(j + 1) * LANES])
        ov[e, :] = acc
        return carry

      lax.fori_loop(0, CH, ebody, 0, unroll=4)

    issue_gather(0, 0)

    def step(i, b):
      wait_gather(b)

      @pl.when(i + 1 < NCH)
      def _():
        issue_gather(i + 1, 1 - b)

      @pl.when(i >= 2)
      def _():
        wait_out(b)

      compute(b)
      issue_out(i, b)

    def jbody(j, carry):
      step(2 * j, 0)
      step(2 * j + 1, 1)
      return carry

    lax.fori_loop(0, NCH // 2, jbody, 0)
    wait_out(0)
    wait_out(1)

  return k(z, src_all, dst_all)


def _loss_body(p_ref, o_ref, *, npos_blocks):
  i = pl.program_id(0)
  x = p_ref[...]
  r = lax.broadcasted_iota(jnp.int32, (DIM, DIM // LANES), 0)
  c = lax.broadcasted_iota(jnp.int32, (DIM, DIM // LANES), 1)
  fold = jnp.where(r // LANES == c, 1.0, 0.0).astype(jnp.float32)
  v = jnp.dot(x, fold, preferred_element_type=jnp.float32)
  s = jax.nn.sigmoid(v)
  arg = jnp.where(i < npos_blocks, s, 1.0 - s) + LOG_EPS
  t = -jnp.log(arg)

  @pl.when(i == 0)
  def _():
    o_ref[...] = jnp.zeros((1, 1), jnp.float32)

  o_ref[...] += jnp.full((1, 1), jnp.sum(t) * (1.0 / NE), jnp.float32)


def _tc_loss(partials):
  total_rows = 2 * NE * LANES // DIM
  rows = 4000
  grid = total_rows // rows
  body = functools.partial(_loss_body, npos_blocks=grid // 2)
  out = pl.pallas_call(
      body,
      out_shape=jax.ShapeDtypeStruct((1, 1), jnp.float32),
      grid=(grid,),
      in_specs=[pl.BlockSpec((rows, DIM), lambda i: (i, 0))],
      out_specs=pl.BlockSpec((1, 1), lambda i: (0, 0)),
  )(partials.reshape(total_rows, DIM))
  return out[0, 0]


def kernel(z, pos_edge_index, neg_edge_index):
  src_all = jnp.concatenate([pos_edge_index[0], neg_edge_index[0]])
  dst_all = jnp.concatenate([pos_edge_index[1], neg_edge_index[1]])
  partials = _sc_partials(z, src_all, dst_all)
  return _tc_loss(partials)


# speedup vs baseline: 8.1791x; 1.8225x over previous
import functools

import jax
import jax.numpy as jnp
from jax import lax
from jax.experimental import pallas as pl
from jax.experimental.pallas import tpu as pltpu
from jax.experimental.pallas import tpu_sc as plsc

LOG_EPS = 1e-8
NE = 320000
DIM = 128
LANES = 16
NWORK = 32
CH = 128
NCH_SIGN = NE // CH
NCH_ALL = 2 * NCH_SIGN
OROW = CH * LANES // DIM
TROW = 2 * NE * LANES // DIM


def _sc_partials(z, pos_edge_index, neg_edge_index):
  mesh = plsc.VectorSubcoreMesh(core_axis_name="c", subcore_axis_name="s")

  @functools.partial(
      pl.kernel,
      out_type=jax.ShapeDtypeStruct((TROW, DIM), jnp.float32),
      mesh=mesh,
      scratch_types=[
          pltpu.VMEM((CH,), jnp.int32),
          pltpu.VMEM((CH,), jnp.int32),
          pltpu.VMEM((CH,), jnp.int32),
          pltpu.VMEM((CH,), jnp.int32),
          pltpu.VMEM((CH, DIM), jnp.float32),
          pltpu.VMEM((CH, DIM), jnp.float32),
          pltpu.VMEM((CH, DIM), jnp.float32),
          pltpu.VMEM((CH, DIM), jnp.float32),
          pltpu.VMEM((OROW, DIM), jnp.float32),
          pltpu.VMEM((OROW, DIM), jnp.float32),
          pltpu.SemaphoreType.DMA,
          pltpu.SemaphoreType.DMA,
          pltpu.SemaphoreType.DMA,
          pltpu.SemaphoreType.DMA,
          pltpu.SemaphoreType.DMA,
          pltpu.SemaphoreType.DMA,
      ],
  )
  def k(z_hbm, pos_hbm, neg_hbm, out_hbm, si0, di0, si1, di1, sr0, dr0, sr1,
        dr1, ov0, ov1, is0, is1, gs0, gs1, os0, os1):
    wid = lax.axis_index("s") * 2 + lax.axis_index("c")
    nch = NCH_ALL // NWORK + jnp.where(wid < NCH_ALL % NWORK, 1, 0)
    sidx = (si0, si1)
    didx = (di0, di1)
    srows = (sr0, sr1)
    drows = (dr0, dr1)
    outv = (ov0, ov1)
    isem = (is0, is1)
    gsem = (gs0, gs1)
    osem = (os0, os1)

    def issue_idx(i, b):
      c = wid + i * NWORK

      @pl.when(c < NCH_SIGN)
      def _():
        off = pl.multiple_of(c * CH, CH)
        pltpu.async_copy(pos_hbm.at[0, pl.ds(off, CH)], sidx[b], isem[b])
        pltpu.async_copy(pos_hbm.at[1, pl.ds(off, CH)], didx[b], isem[b])

      @pl.when(c >= NCH_SIGN)
      def _():
        off = pl.multiple_of((c - NCH_SIGN) * CH, CH)
        pltpu.async_copy(neg_hbm.at[0, pl.ds(off, CH)], sidx[b], isem[b])
        pltpu.async_copy(neg_hbm.at[1, pl.ds(off, CH)], didx[b], isem[b])

    def wait_idx(b):
      pltpu.make_async_copy(pos_hbm.at[0, pl.ds(0, CH)], sidx[b],
                            isem[b]).wait()
      pltpu.make_async_copy(pos_hbm.at[1, pl.ds(0, CH)], didx[b],
                            isem[b]).wait()

    def issue_gather(b):
      pltpu.async_copy(z_hbm.at[sidx[b]], srows[b], gsem[b])
      pltpu.async_copy(z_hbm.at[didx[b]], drows[b], gsem[b])

    def wait_gather(b):
      pltpu.make_async_copy(z_hbm.at[sidx[b]], srows[b], gsem[b]).wait()
      pltpu.make_async_copy(z_hbm.at[didx[b]], drows[b], gsem[b]).wait()

    def issue_out(i, b):
      c = wid + i * NWORK
      rb = pl.multiple_of(c * OROW, OROW)
      pltpu.async_copy(outv[b], out_hbm.at[pl.ds(rb, OROW)], osem[b])

    def wait_out(b):
      pltpu.make_async_copy(outv[b], out_hbm.at[pl.ds(0, OROW)],
                            osem[b]).wait()

    def compute(b):
      sr, dr, ov = srows[b], drows[b], outv[b]

      def rbody(r, carry):
        for q in range(DIM // LANES):
          e = r * 8 + q
          acc = sr[e, 0:LANES] * dr[e, 0:LANES]
          for j in range(1, DIM // LANES):
            acc = acc + (sr[e, j * LANES:(j + 1) * LANES] *
                         dr[e, j * LANES:(j + 1) * LANES])
          ov[r, q * LANES:(q + 1) * LANES] = acc
        return carry

      lax.fori_loop(0, OROW, rbody, 0)

    issue_idx(0, 0)
    issue_idx(1, 1)
    wait_idx(0)
    issue_gather(0)

    def step(i, b):
      @pl.when(i < nch)
      def _():
        wait_gather(b)

        @pl.when(i + 1 < nch)
        def _():
          wait_idx(1 - b)
          issue_gather(1 - b)

        @pl.when(i + 2 < nch)
        def _():
          issue_idx(i + 2, b)

        @pl.when(i >= 2)
        def _():
          wait_out(b)

        compute(b)
        issue_out(i, b)

    def jbody(j, carry):
      step(2 * j, 0)
      step(2 * j + 1, 1)
      return carry

    lax.fori_loop(0, (NCH_ALL // NWORK + 1 + 1) // 2, jbody, 0)
    wait_out(0)
    wait_out(1)

  return k(z, pos_edge_index, neg_edge_index)


def _loss_body(p_ref, o_ref, *, npos_blocks):
  i = pl.program_id(0)
  x = p_ref[...]
  r = lax.broadcasted_iota(jnp.int32, (DIM, DIM // LANES), 0)
  c = lax.broadcasted_iota(jnp.int32, (DIM, DIM // LANES), 1)
  fold = jnp.where(r // LANES == c, 1.0, 0.0).astype(jnp.float32)
  v = jnp.dot(x, fold, preferred_element_type=jnp.float32)
  s = jax.nn.sigmoid(v)
  arg = jnp.where(i < npos_blocks, s, 1.0 - s) + LOG_EPS
  t = -jnp.log(arg)

  @pl.when(i == 0)
  def _():
    o_ref[...] = jnp.zeros((1, 1), jnp.float32)

  o_ref[...] += jnp.full((1, 1), jnp.sum(t) * (1.0 / NE), jnp.float32)


def _tc_loss(partials):
  rows = 4000
  grid = TROW // rows
  body = functools.partial(_loss_body, npos_blocks=grid // 2)
  out = pl.pallas_call(
      body,
      out_shape=jax.ShapeDtypeStruct((1, 1), jnp.float32),
      grid=(grid,),
      in_specs=[pl.BlockSpec((rows, DIM), lambda i: (i, 0))],
      out_specs=pl.BlockSpec((1, 1), lambda i: (0, 0)),
  )(partials)
  return out[0, 0]


def kernel(z, pos_edge_index, neg_edge_index):
  partials = _sc_partials(z, pos_edge_index, neg_edge_index)
  return _tc_loss(partials)


# speedup vs baseline: 8.8173x; 1.0780x over previous
import functools

import jax
import jax.numpy as jnp
from jax import lax
from jax.experimental import pallas as pl
from jax.experimental.pallas import tpu as pltpu
from jax.experimental.pallas import tpu_sc as plsc

LOG_EPS = 1e-8
NE = 320000
DIM = 128
LANES = 16
NWORK = 32
CH = 128
NCH_SIGN = NE // CH
NCH_ALL = 2 * NCH_SIGN
W32 = DIM // 2
OROW = CH * LANES // DIM
TROW = 2 * NE * LANES // DIM


def _sc_partials(z, pos_edge_index, neg_edge_index):
  mesh = plsc.VectorSubcoreMesh(core_axis_name="c", subcore_axis_name="s")

  @functools.partial(
      pl.kernel,
      out_type=jax.ShapeDtypeStruct((TROW, DIM), jnp.float32),
      compiler_params=pltpu.CompilerParams(use_tc_tiling_on_sc=False),
      mesh=mesh,
      scratch_types=[
          pltpu.VMEM((CH,), jnp.int32),
          pltpu.VMEM((CH,), jnp.int32),
          pltpu.VMEM((CH,), jnp.int32),
          pltpu.VMEM((CH,), jnp.int32),
          pltpu.VMEM((CH, W32), jnp.uint32),
          pltpu.VMEM((CH, W32), jnp.uint32),
          pltpu.VMEM((CH, W32), jnp.uint32),
          pltpu.VMEM((CH, W32), jnp.uint32),
          pltpu.VMEM((OROW, DIM), jnp.float32),
          pltpu.VMEM((OROW, DIM), jnp.float32),
          pltpu.SemaphoreType.DMA,
          pltpu.SemaphoreType.DMA,
          pltpu.SemaphoreType.DMA,
          pltpu.SemaphoreType.DMA,
          pltpu.SemaphoreType.DMA,
          pltpu.SemaphoreType.DMA,
      ],
  )
  def k(z_hbm, pos_hbm, neg_hbm, out_hbm, si0, di0, si1, di1, sr0, dr0, sr1,
        dr1, ov0, ov1, is0, is1, gs0, gs1, os0, os1):
    wid = lax.axis_index("s") * 2 + lax.axis_index("c")
    nch = NCH_ALL // NWORK + jnp.where(wid < NCH_ALL % NWORK, 1, 0)
    sidx = (si0, si1)
    didx = (di0, di1)
    srows = (sr0, sr1)
    drows = (dr0, dr1)
    outv = (ov0, ov1)
    isem = (is0, is1)
    gsem = (gs0, gs1)
    osem = (os0, os1)

    def issue_idx(i, b):
      c = wid + i * NWORK

      @pl.when(c < NCH_SIGN)
      def _():
        off = pl.multiple_of(c * CH, CH)
        pltpu.async_copy(pos_hbm.at[0, pl.ds(off, CH)], sidx[b], isem[b])
        pltpu.async_copy(pos_hbm.at[1, pl.ds(off, CH)], didx[b], isem[b])

      @pl.when(c >= NCH_SIGN)
      def _():
        off = pl.multiple_of((c - NCH_SIGN) * CH, CH)
        pltpu.async_copy(neg_hbm.at[0, pl.ds(off, CH)], sidx[b], isem[b])
        pltpu.async_copy(neg_hbm.at[1, pl.ds(off, CH)], didx[b], isem[b])

    def wait_idx(b):
      pltpu.make_async_copy(pos_hbm.at[0, pl.ds(0, CH)], sidx[b],
                            isem[b]).wait()
      pltpu.make_async_copy(pos_hbm.at[1, pl.ds(0, CH)], didx[b],
                            isem[b]).wait()

    def issue_gather(b):
      pltpu.async_copy(z_hbm.at[sidx[b]], srows[b], gsem[b])
      pltpu.async_copy(z_hbm.at[didx[b]], drows[b], gsem[b])

    def wait_gather(b):
      pltpu.make_async_copy(z_hbm.at[sidx[b]], srows[b], gsem[b]).wait()
      pltpu.make_async_copy(z_hbm.at[didx[b]], drows[b], gsem[b]).wait()

    def issue_out(i, b):
      c = wid + i * NWORK
      rb = pl.multiple_of(c * OROW, OROW)
      pltpu.async_copy(outv[b], out_hbm.at[pl.ds(rb, OROW)], osem[b])

    def wait_out(b):
      pltpu.make_async_copy(outv[b], out_hbm.at[pl.ds(0, OROW)],
                            osem[b]).wait()

    def compute(b):
      sr, dr, ov = srows[b], drows[b], outv[b]

      def rbody(r, carry):
        for q in range(DIM // LANES):
          e = r * 8 + q
          acc = None
          for j in range(W32 // LANES):
            sv = sr[e, j * LANES:(j + 1) * LANES]
            dv = dr[e, j * LANES:(j + 1) * LANES]
            s0 = lax.bitcast_convert_type(sv << 16, jnp.float32)
            s1 = lax.bitcast_convert_type(sv & jnp.uint32(0xFFFF0000),
                                          jnp.float32)
            d0 = lax.bitcast_convert_type(dv << 16, jnp.float32)
            d1 = lax.bitcast_convert_type(dv & jnp.uint32(0xFFFF0000),
                                          jnp.float32)
            term = s0 * d0 + s1 * d1
            acc = term if acc is None else acc + term
          ov[r, q * LANES:(q + 1) * LANES] = acc
        return carry

      lax.fori_loop(0, OROW, rbody, 0)

    issue_idx(0, 0)
    issue_idx(1, 1)
    wait_idx(0)
    issue_gather(0)

    def step(i, b):
      @pl.when(i < nch)
      def _():
        wait_gather(b)

        @pl.when(i + 1 < nch)
        def _():
          wait_idx(1 - b)
          issue_gather(1 - b)

        @pl.when(i + 2 < nch)
        def _():
          issue_idx(i + 2, b)

        @pl.when(i >= 2)
        def _():
          wait_out(b)

        compute(b)
        issue_out(i, b)

    def jbody(j, carry):
      step(2 * j, 0)
      step(2 * j + 1, 1)
      return carry

    lax.fori_loop(0, (NCH_ALL // NWORK + 1 + 1) // 2, jbody, 0)
    wait_out(0)
    wait_out(1)

  return k(z, pos_edge_index, neg_edge_index)


def _loss_body(p_ref, o_ref, *, npos_blocks):
  i = pl.program_id(0)
  x = p_ref[...]
  r = lax.broadcasted_iota(jnp.int32, (DIM, DIM // LANES), 0)
  c = lax.broadcasted_iota(jnp.int32, (DIM, DIM // LANES), 1)
  fold = jnp.where(r // LANES == c, 1.0, 0.0).astype(jnp.float32)
  v = jnp.dot(x, fold, preferred_element_type=jnp.float32)
  s = jax.nn.sigmoid(v)
  arg = jnp.where(i < npos_blocks, s, 1.0 - s) + LOG_EPS
  t = -jnp.log(arg)

  @pl.when(i == 0)
  def _():
    o_ref[...] = jnp.zeros((1, 1), jnp.float32)

  o_ref[...] += jnp.full((1, 1), jnp.sum(t) * (1.0 / NE), jnp.float32)


def _tc_loss(partials):
  rows = 4000
  grid = TROW // rows
  body = functools.partial(_loss_body, npos_blocks=grid // 2)
  out = pl.pallas_call(
      body,
      out_shape=jax.ShapeDtypeStruct((1, 1), jnp.float32),
      grid=(grid,),
      in_specs=[pl.BlockSpec((rows, DIM), lambda i: (i, 0))],
      out_specs=pl.BlockSpec((1, 1), lambda i: (0, 0)),
  )(partials)
  return out[0, 0]


def kernel(z, pos_edge_index, neg_edge_index):
  z16 = z.astype(jnp.bfloat16)
  zp = lax.bitcast_convert_type(z16.reshape(z.shape[0], W32, 2), jnp.uint32)
  partials = _sc_partials(zp, pos_edge_index, neg_edge_index)
  return _tc_loss(partials)


# speedup vs baseline: 9.6761x; 1.0974x over previous
import functools

import jax
import jax.numpy as jnp
from jax import lax
from jax.experimental import pallas as pl
from jax.experimental.pallas import tpu as pltpu
from jax.experimental.pallas import tpu_sc as plsc

LOG_EPS = 1e-8
NE = 320000
DIM = 128
LANES = 16
NWORK = 32
CH = 128
NCH_SIGN = NE // CH
NCH_ALL = 2 * NCH_SIGN
W32 = DIM // 2
OROW = CH * LANES // DIM
TROW = 2 * NE * LANES // DIM


def _sc_partials(z, pos_edge_index, neg_edge_index):
  mesh = plsc.VectorSubcoreMesh(core_axis_name="c", subcore_axis_name="s")

  @functools.partial(
      pl.kernel,
      out_type=jax.ShapeDtypeStruct((TROW, DIM), jnp.float32),
      compiler_params=pltpu.CompilerParams(use_tc_tiling_on_sc=False),
      mesh=mesh,
      scratch_types=[
          pltpu.VMEM((CH,), jnp.int32),
          pltpu.VMEM((CH,), jnp.int32),
          pltpu.VMEM((CH,), jnp.int32),
          pltpu.VMEM((CH,), jnp.int32),
          pltpu.VMEM((CH, W32), jnp.uint32),
          pltpu.VMEM((CH, W32), jnp.uint32),
          pltpu.VMEM((CH, W32), jnp.uint32),
          pltpu.VMEM((CH, W32), jnp.uint32),
          pltpu.VMEM((OROW, DIM), jnp.float32),
          pltpu.VMEM((OROW, DIM), jnp.float32),
          pltpu.SemaphoreType.DMA,
          pltpu.SemaphoreType.DMA,
          pltpu.SemaphoreType.DMA,
          pltpu.SemaphoreType.DMA,
          pltpu.SemaphoreType.DMA,
          pltpu.SemaphoreType.DMA,
      ],
  )
  def k(z_hbm, pos_hbm, neg_hbm, out_hbm, si0, di0, si1, di1, sr0, dr0, sr1,
        dr1, ov0, ov1, is0, is1, gs0, gs1, os0, os1):
    wid = lax.axis_index("s") * 2 + lax.axis_index("c")
    nch = NCH_ALL // NWORK + jnp.where(wid < NCH_ALL % NWORK, 1, 0)
    sidx = (si0, si1)
    didx = (di0, di1)
    srows = (sr0, sr1)
    drows = (dr0, dr1)
    outv = (ov0, ov1)
    isem = (is0, is1)
    gsem = (gs0, gs1)
    osem = (os0, os1)

    def issue_idx(i, b):
      c = wid + i * NWORK

      @pl.when(c < NCH_SIGN)
      def _():
        off = pl.multiple_of(c * CH, CH)
        pltpu.async_copy(pos_hbm.at[0, pl.ds(off, CH)], sidx[b], isem[b])
        pltpu.async_copy(pos_hbm.at[1, pl.ds(off, CH)], didx[b], isem[b])

      @pl.when(c >= NCH_SIGN)
      def _():
        off = pl.multiple_of((c - NCH_SIGN) * CH, CH)
        pltpu.async_copy(neg_hbm.at[0, pl.ds(off, CH)], sidx[b], isem[b])
        pltpu.async_copy(neg_hbm.at[1, pl.ds(off, CH)], didx[b], isem[b])

    def wait_idx(b):
      pltpu.make_async_copy(pos_hbm.at[0, pl.ds(0, CH)], sidx[b],
                            isem[b]).wait()
      pltpu.make_async_copy(pos_hbm.at[1, pl.ds(0, CH)], didx[b],
                            isem[b]).wait()

    def issue_gather(b):
      pltpu.async_copy(z_hbm.at[sidx[b]], srows[b], gsem[b])
      pltpu.async_copy(z_hbm.at[didx[b]], drows[b], gsem[b])

    def wait_gather(b):
      pltpu.make_async_copy(z_hbm.at[sidx[b]], srows[b], gsem[b]).wait()
      pltpu.make_async_copy(z_hbm.at[didx[b]], drows[b], gsem[b]).wait()

    def issue_out(i, b):
      c = wid + i * NWORK
      rb = pl.multiple_of(c * OROW, OROW)
      pltpu.async_copy(outv[b], out_hbm.at[pl.ds(rb, OROW)], osem[b])

    def wait_out(b):
      pltpu.make_async_copy(outv[b], out_hbm.at[pl.ds(0, OROW)],
                            osem[b]).wait()

    def compute(b):
      sr, dr, ov = srows[b], drows[b], outv[b]

      def rbody(r, carry):
        for q in range(DIM // LANES):
          e = r * 8 + q
          acc = None
          for j in range(W32 // LANES):
            sv = sr[e, j * LANES:(j + 1) * LANES]
            dv = dr[e, j * LANES:(j + 1) * LANES]
            s0 = lax.bitcast_convert_type(sv << 16, jnp.float32)
            s1 = lax.bitcast_convert_type(sv, jnp.float32)
            d0 = lax.bitcast_convert_type(dv << 16, jnp.float32)
            d1 = lax.bitcast_convert_type(dv, jnp.float32)
            term = s0 * d0 + s1 * d1
            acc = term if acc is None else acc + term
          ov[r, q * LANES:(q + 1) * LANES] = acc
        return carry

      lax.fori_loop(0, OROW, rbody, 0)

    issue_idx(0, 0)
    issue_idx(1, 1)
    wait_idx(0)
    issue_gather(0)

    def step(i, b):
      @pl.when(i < nch)
      def _():
        wait_gather(b)

        @pl.when(i + 1 < nch)
        def _():
          wait_idx(1 - b)
          issue_gather(1 - b)

        @pl.when(i + 2 < nch)
        def _():
          issue_idx(i + 2, b)

        @pl.when(i >= 2)
        def _():
          wait_out(b)

        compute(b)
        issue_out(i, b)

    def jbody(j, carry):
      step(2 * j, 0)
      step(2 * j + 1, 1)
      return carry

    lax.fori_loop(0, (NCH_ALL // NWORK + 1 + 1) // 2, jbody, 0)
    wait_out(0)
    wait_out(1)

  return k(z, pos_edge_index, neg_edge_index)


def _loss_body(p_ref, o_ref, *, npos_blocks):
  i = pl.program_id(0)
  x = p_ref[...]
  r = lax.broadcasted_iota(jnp.int32, (DIM, DIM // LANES), 0)
  c = lax.broadcasted_iota(jnp.int32, (DIM, DIM // LANES), 1)
  fold = jnp.where(r // LANES == c, 1.0, 0.0).astype(jnp.float32)
  v = jnp.dot(x, fold, preferred_element_type=jnp.float32)
  s = jax.nn.sigmoid(v)
  arg = jnp.where(i < npos_blocks, s, 1.0 - s) + LOG_EPS
  t = -jnp.log(arg)

  @pl.when(i == 0)
  def _():
    o_ref[...] = jnp.zeros((1, 1), jnp.float32)

  o_ref[...] += jnp.full((1, 1), jnp.sum(t) * (1.0 / NE), jnp.float32)


def _tc_loss(partials):
  rows = 4000
  grid = TROW // rows
  body = functools.partial(_loss_body, npos_blocks=grid // 2)
  out = pl.pallas_call(
      body,
      out_shape=jax.ShapeDtypeStruct((1, 1), jnp.float32),
      grid=(grid,),
      in_specs=[pl.BlockSpec((rows, DIM), lambda i: (i, 0))],
      out_specs=pl.BlockSpec((1, 1), lambda i: (0, 0)),
  )(partials)
  return out[0, 0]


def kernel(z, pos_edge_index, neg_edge_index):
  z16 = z.astype(jnp.bfloat16)
  zp = lax.bitcast_convert_type(z16.reshape(z.shape[0], W32, 2), jnp.uint32)
  partials = _sc_partials(zp, pos_edge_index, neg_edge_index)
  return _tc_loss(partials)


# speedup vs baseline: 10.2874x; 1.0632x over previous
import functools

import jax
import jax.numpy as jnp
from jax import lax
from jax.experimental import pallas as pl
from jax.experimental.pallas import tpu as pltpu
from jax.experimental.pallas import tpu_sc as plsc

LOG_EPS = 1e-8
NE = 320000
DIM = 128
LANES = 16
NWORK = 32
CH = 128
NCH_SIGN = NE // CH
NCH_ALL = 2 * NCH_SIGN
W32 = DIM // 2
OROW = CH * LANES // DIM
TROW = 2 * NE * LANES // DIM


def _sc_partials(z, pos_edge_index, neg_edge_index):
  mesh = plsc.VectorSubcoreMesh(core_axis_name="c", subcore_axis_name="s")

  @functools.partial(
      pl.kernel,
      out_type=jax.ShapeDtypeStruct((TROW, DIM), jnp.float32),
      compiler_params=pltpu.CompilerParams(use_tc_tiling_on_sc=False),
      mesh=mesh,
      scratch_types=[
          pltpu.VMEM((CH,), jnp.int32),
          pltpu.VMEM((CH,), jnp.int32),
          pltpu.VMEM((CH,), jnp.int32),
          pltpu.VMEM((CH,), jnp.int32),
          pltpu.VMEM((CH, W32), jnp.uint32),
          pltpu.VMEM((CH, W32), jnp.uint32),
          pltpu.VMEM((CH, W32), jnp.uint32),
          pltpu.VMEM((CH, W32), jnp.uint32),
          pltpu.VMEM((OROW, DIM), jnp.float32),
          pltpu.VMEM((OROW, DIM), jnp.float32),
          pltpu.SemaphoreType.DMA,
          pltpu.SemaphoreType.DMA,
          pltpu.SemaphoreType.DMA,
          pltpu.SemaphoreType.DMA,
          pltpu.SemaphoreType.DMA,
          pltpu.SemaphoreType.DMA,
      ],
  )
  def k(z_hbm, pos_hbm, neg_hbm, out_hbm, si0, di0, si1, di1, sr0, dr0, sr1,
        dr1, ov0, ov1, is0, is1, gs0, gs1, os0, os1):
    wid = lax.axis_index("s") * 2 + lax.axis_index("c")
    nch = NCH_ALL // NWORK + jnp.where(wid < NCH_ALL % NWORK, 1, 0)
    sidx = (si0, si1)
    didx = (di0, di1)
    srows = (sr0, sr1)
    drows = (dr0, dr1)
    outv = (ov0, ov1)
    isem = (is0, is1)
    gsem = (gs0, gs1)
    osem = (os0, os1)

    def issue_idx(i, b):
      c = wid + i * NWORK

      @pl.when(c < NCH_SIGN)
      def _():
        off = pl.multiple_of(c * CH, CH)
        pltpu.async_copy(pos_hbm.at[0, pl.ds(off, CH)], sidx[b], isem[b])
        pltpu.async_copy(pos_hbm.at[1, pl.ds(off, CH)], didx[b], isem[b])

      @pl.when(c >= NCH_SIGN)
      def _():
        off = pl.multiple_of((c - NCH_SIGN) * CH, CH)
        pltpu.async_copy(neg_hbm.at[0, pl.ds(off, CH)], sidx[b], isem[b])
        pltpu.async_copy(neg_hbm.at[1, pl.ds(off, CH)], didx[b], isem[b])

    def wait_idx(b):
      pltpu.make_async_copy(pos_hbm.at[0, pl.ds(0, CH)], sidx[b],
                            isem[b]).wait()
      pltpu.make_async_copy(pos_hbm.at[1, pl.ds(0, CH)], didx[b],
                            isem[b]).wait()

    def issue_gather(b):
      pltpu.async_copy(z_hbm.at[sidx[b]], srows[b], gsem[b])
      pltpu.async_copy(z_hbm.at[didx[b]], drows[b], gsem[b])

    def wait_gather(b):
      pltpu.make_async_copy(z_hbm.at[sidx[b]], srows[b], gsem[b]).wait()
      pltpu.make_async_copy(z_hbm.at[didx[b]], drows[b], gsem[b]).wait()

    def issue_out(i, b):
      c = wid + i * NWORK
      rb = pl.multiple_of(c * OROW, OROW)
      pltpu.async_copy(outv[b], out_hbm.at[pl.ds(rb, OROW)], osem[b])

    def wait_out(b):
      pltpu.make_async_copy(outv[b], out_hbm.at[pl.ds(0, OROW)],
                            osem[b]).wait()

    def compute(b):
      sr, dr, ov = srows[b], drows[b], outv[b]

      def rbody(r, carry):
        for q in range(DIM // LANES):
          e = r * 8 + q
          acc = None
          for j in range(W32 // LANES):
            sv = sr[e, j * LANES:(j + 1) * LANES]
            dv = dr[e, j * LANES:(j + 1) * LANES]
            s0 = lax.bitcast_convert_type(sv << 16, jnp.float32)
            s1 = lax.bitcast_convert_type(sv, jnp.float32)
            d0 = lax.bitcast_convert_type(dv << 16, jnp.float32)
            d1 = lax.bitcast_convert_type(dv, jnp.float32)
            term = s0 * d0 + s1 * d1
            acc = term if acc is None else acc + term
          ov[r, q * LANES:(q + 1) * LANES] = acc
        return carry

      lax.fori_loop(0, OROW, rbody, 0, unroll=2)

    issue_idx(0, 0)
    issue_idx(1, 1)
    wait_idx(0)
    issue_gather(0)

    def step(i, b):
      @pl.when(i < nch)
      def _():
        wait_gather(b)

        @pl.when(i + 1 < nch)
        def _():
          wait_idx(1 - b)
          issue_gather(1 - b)

        @pl.when(i + 2 < nch)
        def _():
          issue_idx(i + 2, b)

        @pl.when(i >= 2)
        def _():
          wait_out(b)

        compute(b)
        issue_out(i, b)

    def jbody(j, carry):
      step(2 * j, 0)
      step(2 * j + 1, 1)
      return carry

    lax.fori_loop(0, (NCH_ALL // NWORK + 1 + 1) // 2, jbody, 0)
    wait_out(0)
    wait_out(1)

  return k(z, pos_edge_index, neg_edge_index)


def _loss_body(p_ref, o_ref, *, npos_blocks):
  i = pl.program_id(0)
  x = p_ref[...]
  r = lax.broadcasted_iota(jnp.int32, (DIM, DIM // LANES), 0)
  c = lax.broadcasted_iota(jnp.int32, (DIM, DIM // LANES), 1)
  fold = jnp.where(r // LANES == c, 1.0, 0.0).astype(jnp.float32)
  v = jnp.dot(x, fold, preferred_element_type=jnp.float32)
  s = jax.nn.sigmoid(v)
  arg = jnp.where(i < npos_blocks, s, 1.0 - s) + LOG_EPS
  t = -jnp.log(arg)

  @pl.when(i == 0)
  def _():
    o_ref[...] = jnp.zeros((1, 1), jnp.float32)

  o_ref[...] += jnp.full((1, 1), jnp.sum(t) * (1.0 / NE), jnp.float32)


def _tc_loss(partials):
  rows = 4000
  grid = TROW // rows
  body = functools.partial(_loss_body, npos_blocks=grid // 2)
  out = pl.pallas_call(
      body,
      out_shape=jax.ShapeDtypeStruct((1, 1), jnp.float32),
      grid=(grid,),
      in_specs=[pl.BlockSpec((rows, DIM), lambda i: (i, 0))],
      out_specs=pl.BlockSpec((1, 1), lambda i: (0, 0)),
  )(partials)
  return out[0, 0]


def _pack_body(z_ref, o_ref):
  t = lax.bitcast_convert_type(z_ref[...], jnp.uint32)
  bf = (t + jnp.uint32(0x7FFF) + ((t >> 16) & jnp.uint32(1))) >> 16
  bff = bf.astype(jnp.float32)
  r = lax.broadcasted_iota(jnp.int32, (DIM, W32), 0)
  c = lax.broadcasted_iota(jnp.int32, (DIM, W32), 1)
  ev_m = (r == 2 * c).astype(jnp.float32)
  od_m = (r == 2 * c + 1).astype(jnp.float32)
  ev = jnp.dot(bff, ev_m, preferred_element_type=jnp.float32).astype(jnp.uint32)
  od = jnp.dot(bff, od_m, preferred_element_type=jnp.float32).astype(jnp.uint32)
  o_ref[...] = ev | (od << 16)


def _pack_z(z):
  return pl.pallas_call(
      _pack_body,
      out_shape=jax.ShapeDtypeStruct((z.shape[0], W32), jnp.uint32),
  )(z)


def kernel(z, pos_edge_index, neg_edge_index):
  partials = _sc_partials(_pack_z(z), pos_edge_index, neg_edge_index)
  return _tc_loss(partials)
